# baseline (speedup 1.0000x reference)
import functools

import numpy as np
import jax
import jax.numpy as jnp
from jax import lax
from jax.experimental import pallas as pl
from jax.experimental.pallas import tpu as pltpu
from jax.experimental.pallas import tpu_sc as plsc

B = 16384
N_COL = 26
VOCAB = 100000
D = 64
ROWS = B * N_COL

NC = 2
NS = 16
L = 16
NW = NC * NS
PER_W = ROWS // NW
UNIT = 128
N_UNITS = PER_W // UNIT


def _make_pe(max_len, d):
    position = np.arange(max_len, dtype=np.float64)[:, None]
    div_term = np.exp(np.arange(0, d, 2, dtype=np.float64) * (-np.log(10000.0) / d))
    pe = np.zeros((max_len, d), dtype=np.float32)
    pe[:, 0::2] = np.sin(position * div_term).astype(np.float32)
    pe[:, 1::2] = np.cos(position * div_term).astype(np.float32)
    return pe


_PE26 = _make_pe(N_COL, D)

_MESH = plsc.VectorSubcoreMesh(core_axis_name="c", subcore_axis_name="s")


@functools.partial(
    pl.kernel,
    mesh=_MESH,
    out_type=jax.ShapeDtypeStruct((ROWS, D), jnp.float32),
    scratch_types=[
        pltpu.VMEM((UNIT,), jnp.float32),
        pltpu.VMEM((UNIT,), jnp.int32),
        pltpu.VMEM((UNIT, D), jnp.float32),
        pltpu.SemaphoreType.DMA,
    ],
    compiler_params=pltpu.CompilerParams(use_tc_tiling_on_sc=False),
)
def _sc_gather(x_hbm, tab_hbm, out_hbm, x_v, idx_v, rows_v, sem):
    wid = lax.axis_index("s") * NC + lax.axis_index("c")
    base = wid * PER_W

    def unit_body(u, carry):
        off = base + u * UNIT
        pltpu.sync_copy(x_hbm.at[pl.ds(off, UNIT)], x_v)
        for g in range(UNIT // L):
            r0 = off + g * L
            col = lax.rem(r0 + lax.iota(jnp.int32, L), N_COL)
            idx_v[pl.ds(g * L, L)] = (
                x_v[pl.ds(g * L, L)].astype(jnp.int32) + col * VOCAB
            )
        pltpu.async_copy(tab_hbm.at[idx_v], rows_v, sem).wait()
        pltpu.sync_copy(rows_v, out_hbm.at[pl.ds(off, UNIT)])
        return carry

    lax.fori_loop(0, N_UNITS, unit_body, 0)


BLK = 26 * 64
GRID = ROWS // BLK


def _tc_body(emb_ref, wt_ref, add_ref, out_ref):
    h = jnp.maximum(emb_ref[...], 0.0)
    out_ref[...] = (
        jnp.dot(h, wt_ref[...], preferred_element_type=jnp.float32) + add_ref[...]
    )


def _tc_dense(emb, wt, add_blk):
    return pl.pallas_call(
        _tc_body,
        grid=(GRID,),
        in_specs=[
            pl.BlockSpec((BLK, D), lambda i: (i, 0)),
            pl.BlockSpec((D, D), lambda i: (0, 0)),
            pl.BlockSpec((BLK, D), lambda i: (0, 0)),
        ],
        out_specs=pl.BlockSpec((BLK, D), lambda i: (i, 0)),
        out_shape=jax.ShapeDtypeStruct((ROWS, D), jnp.float32),
    )(emb, wt, add_blk)


def kernel(x, tables, W, b):
    x_flat = x.reshape(ROWS)
    tab_flat = tables.reshape(N_COL * VOCAB, D)
    emb = _sc_gather(x_flat, tab_flat)
    add_blk = jnp.tile(jnp.asarray(_PE26) + b[None, :], (BLK // N_COL, 1))
    out = _tc_dense(emb, W.T, add_blk)
    return out.reshape(B, N_COL, D)


# speedup vs baseline: 2.3601x; 2.3601x over previous
import functools

import numpy as np
import jax
import jax.numpy as jnp
from jax import lax
from jax.experimental import pallas as pl
from jax.experimental.pallas import tpu as pltpu
from jax.experimental.pallas import tpu_sc as plsc

B = 16384
N_COL = 26
VOCAB = 100000
D = 64
ROWS = N_COL * D

NC = 2
NS = 16
L = 16
NW = NC * NS
D_PER_W = D // NW

OUT_CHUNK = 2048
N_CHUNK = B // OUT_CHUNK
G_PER_CHUNK = OUT_CHUNK // L


def _make_pe(max_len, d):
    position = np.arange(max_len, dtype=np.float64)[:, None]
    div_term = np.exp(np.arange(0, d, 2, dtype=np.float64) * (-np.log(10000.0) / d))
    pe = np.zeros((max_len, d), dtype=np.float32)
    pe[:, 0::2] = np.sin(position * div_term).astype(np.float32)
    pe[:, 1::2] = np.cos(position * div_term).astype(np.float32)
    return pe


_PE26 = _make_pe(N_COL, D)

_MESH = plsc.VectorSubcoreMesh(core_axis_name="c", subcore_axis_name="s")


@functools.partial(
    pl.kernel,
    mesh=_MESH,
    out_type=jax.ShapeDtypeStruct((ROWS, B), jnp.float32),
    scratch_types=[
        pltpu.VMEM((B,), jnp.int32),
        pltpu.VMEM((VOCAB,), jnp.float32),
        pltpu.VMEM((OUT_CHUNK,), jnp.float32),
        pltpu.SemaphoreType.DMA,
    ],
    compiler_params=pltpu.CompilerParams(needs_layout_passes=False),
)
def _sc_gather(idx_hbm, tab_hbm, out_hbm, idx_v, vocab_v, out_v, sem):
    wid = lax.axis_index("s") * NC + lax.axis_index("c")

    def col_body(c, carry):
        row0 = c * D + wid * D_PER_W
        vdma = pltpu.async_copy(tab_hbm.at[row0], vocab_v, sem)
        pltpu.sync_copy(idx_hbm.at[c], idx_v)
        vdma.wait()

        def row_body(rb, row):
            def chunk_body(j, carry2):
                base = j * OUT_CHUNK

                def g_body(g, carry3):
                    o = g * L
                    iv = idx_v[pl.ds(base + o, L)]
                    out_v[pl.ds(o, L)] = plsc.load_gather(vocab_v, [iv])
                    return carry3

                lax.fori_loop(0, G_PER_CHUNK, g_body, 0)
                pltpu.sync_copy(out_v, out_hbm.at[row, pl.ds(base, OUT_CHUNK)])
                return carry2

            lax.fori_loop(0, N_CHUNK, chunk_body, 0)
            return row

        for rb in range(D_PER_W):
            row = row0 + rb
            if rb > 0:
                pltpu.sync_copy(tab_hbm.at[row], vocab_v)
            row_body(rb, row)
        return carry

    lax.fori_loop(0, N_COL, col_body, 0)


BC = 2048
NBC = B // BC


def _tc_body(emb_ref, w_ref, add_ref, out_ref):
    h = jnp.maximum(emb_ref[0], 0.0)
    out_ref[0] = (
        jnp.dot(w_ref[...], h, preferred_element_type=jnp.float32) + add_ref[0]
    )


def _tc_dense(emb_t, w, add3):
    return pl.pallas_call(
        _tc_body,
        grid=(N_COL, NBC),
        in_specs=[
            pl.BlockSpec((1, D, BC), lambda c, j: (c, 0, j)),
            pl.BlockSpec((D, D), lambda c, j: (0, 0)),
            pl.BlockSpec((1, D, 1), lambda c, j: (c, 0, 0)),
        ],
        out_specs=pl.BlockSpec((1, D, BC), lambda c, j: (c, 0, j)),
        out_shape=jax.ShapeDtypeStruct((N_COL, D, B), jnp.float32),
    )(emb_t, w, add3)


def kernel(x, tables, W, b):
    idx_t = x.T.astype(jnp.int32)
    tab_t = tables.transpose(0, 2, 1).reshape(ROWS, VOCAB)
    emb_t = _sc_gather(idx_t, tab_t)
    add3 = (jnp.asarray(_PE26) + b[None, :])[:, :, None]
    out_t = _tc_dense(emb_t.reshape(N_COL, D, B), W, add3)
    return out_t.transpose(2, 0, 1)
